# baseline (speedup 1.0000x reference)
import functools

import jax
import jax.numpy as jnp
from jax import lax
from jax.experimental import pallas as pl
from jax.experimental.pallas import tpu as pltpu
from jax.experimental.pallas import tpu_sc as plsc

N_RAW = 10000
N_PAD = 10240
E_RAW = 320000
CH = 128
N_TILES = 32
CPT = 80
E_PAD = CPT * N_TILES * CH
ROWS_PER_TILE = N_PAD // 16
NG = 64
F = 128
BLK = 1024
GRID = N_PAD // BLK

_mesh = plsc.VectorSubcoreMesh(core_axis_name="c", subcore_axis_name="s")


@functools.partial(
    pl.kernel,
    mesh=_mesh,
    out_type=jax.ShapeDtypeStruct((2, N_PAD, 16), jnp.float32),
    scratch_types=[
        pltpu.VMEM((CPT, CH), jnp.int32),
        pltpu.VMEM((CH, 16), jnp.float32),
        pltpu.VMEM_SHARED((N_PAD, 16), jnp.float32),
    ],
)
def _sc_deg(dst_hbm, zeros16_hbm, out_hbm, dbuf, ones_v, acc):
    cid = lax.axis_index("c")
    sid = lax.axis_index("s")
    wid = sid * 2 + cid

    def fill(r, _):
        ones_v[r, :] = jnp.ones((16,), jnp.float32)
        return 0

    lax.fori_loop(0, CH, fill, 0)

    r0 = sid * ROWS_PER_TILE
    pltpu.sync_copy(zeros16_hbm.at[pl.ds(r0, ROWS_PER_TILE)],
                    acc.at[pl.ds(r0, ROWS_PER_TILE)])
    pltpu.sync_copy(dst_hbm.at[pl.ds(wid * CPT, CPT)], dbuf)
    plsc.subcore_barrier()

    def chunk(j, _):
        pltpu.sync_copy(ones_v, acc.at[dbuf.at[j]], add=True)
        return 0

    lax.fori_loop(0, CPT, chunk, 0)
    plsc.subcore_barrier()

    pltpu.sync_copy(acc.at[pl.ds(r0, ROWS_PER_TILE)],
                    out_hbm.at[cid, pl.ds(r0, ROWS_PER_TILE)])


@functools.partial(
    pl.kernel,
    mesh=_mesh,
    out_type=jax.ShapeDtypeStruct((2, N_PAD, F), jnp.float32),
    scratch_types=[
        pltpu.VMEM((CH,), jnp.int32),
        pltpu.VMEM((CH,), jnp.int32),
        pltpu.VMEM((CH, F), jnp.float32),
        pltpu.VMEM_SHARED((N_PAD, F), jnp.float32),
        pltpu.SemaphoreType.DMA,
    ],
)
def _sc_agg(src_hbm, dst_hbm, g_hbm, zeros_hbm, out_hbm,
            sidx, didx, rows, acc, semg):
    cid = lax.axis_index("c")
    sid = lax.axis_index("s")

    r0 = sid * ROWS_PER_TILE
    pltpu.sync_copy(zeros_hbm.at[pl.ds(r0, ROWS_PER_TILE)],
                    acc.at[pl.ds(r0, ROWS_PER_TILE)])
    plsc.subcore_barrier()

    wid = sid * 2 + cid
    base = wid * CPT * CH

    def chunk(j, _):
        off = pl.multiple_of(base + j * CH, CH)
        pltpu.sync_copy(src_hbm.at[pl.ds(off, CH)], sidx)
        pltpu.sync_copy(dst_hbm.at[pl.ds(off, CH)], didx)
        pltpu.async_copy(g_hbm.at[sidx], rows, semg).wait()
        pltpu.sync_copy(rows, acc.at[didx], add=True)
        return 0

    lax.fori_loop(0, CPT, chunk, 0)
    plsc.subcore_barrier()

    pltpu.sync_copy(acc.at[pl.ds(r0, ROWS_PER_TILE)],
                    out_hbm.at[cid, pl.ds(r0, ROWS_PER_TILE)])


def _dinv_of(degp_blk):
    deg = degp_blk[0] + degp_blk[1] + 1.0
    return lax.rsqrt(deg[:, :1])


def _tc_in_body(x_ref, degp_ref, w_ref, g_ref):
    dinv = _dinv_of(degp_ref[...])
    h = jnp.dot(x_ref[...], w_ref[...], preferred_element_type=jnp.float32)
    g_ref[...] = h * dinv


def _tc_mid_body(ag_ref, g_ref, degp_ref, b_ref, w_ref, out_ref):
    dinv = _dinv_of(degp_ref[...])
    ag = ag_ref[...]
    a = (ag[0] + ag[1] + g_ref[...]) * dinv + b_ref[...]
    h = jnp.maximum(a, 0.0)
    out_ref[...] = jnp.dot(h, w_ref[...],
                           preferred_element_type=jnp.float32) * dinv


def _tc_fin_body(ag_ref, g_ref, degp_ref, b_ref, batch_ref, wl_ref, bl_ref,
                 out_ref, acc_s, acc_c):
    i = pl.program_id(0)

    @pl.when(i == 0)
    def _():
        acc_s[...] = jnp.zeros_like(acc_s)
        acc_c[...] = jnp.zeros_like(acc_c)

    dinv = _dinv_of(degp_ref[...])
    ag = ag_ref[...]
    a = (ag[0] + ag[1] + g_ref[...]) * dinv + b_ref[...]
    h = jnp.maximum(a, 0.0)
    gids = lax.broadcasted_iota(jnp.int32, (BLK, NG), 1)
    onehot = (batch_ref[...] == gids).astype(jnp.float32)
    acc_s[...] = acc_s[...] + lax.dot_general(
        onehot, h, (((0,), (0,)), ((), ())),
        preferred_element_type=jnp.float32)
    acc_c[...] = acc_c[...] + jnp.sum(onehot, axis=0)[:, None]

    @pl.when(i == GRID - 1)
    def _():
        pooled = acc_s[...] / jnp.maximum(acc_c[...], 1.0)
        out_ref[...] = jnp.dot(pooled, wl_ref[...],
                               preferred_element_type=jnp.float32) + bl_ref[...]


_degp_spec = pl.BlockSpec((2, BLK, 16), lambda i: (0, i, 0))
_row_spec = pl.BlockSpec((BLK, F), lambda i: (i, 0))
_ag_spec = pl.BlockSpec((2, BLK, F), lambda i: (0, i, 0))
_w_spec = pl.BlockSpec((F, F), lambda i: (0, 0))
_b_spec = pl.BlockSpec((1, F), lambda i: (0, 0))

_tc_in = pl.pallas_call(
    _tc_in_body,
    grid=(GRID,),
    in_specs=[_row_spec, _degp_spec, _w_spec],
    out_specs=_row_spec,
    out_shape=jax.ShapeDtypeStruct((N_PAD, F), jnp.float32),
)

_tc_mid = pl.pallas_call(
    _tc_mid_body,
    grid=(GRID,),
    in_specs=[_ag_spec, _row_spec, _degp_spec, _b_spec, _w_spec],
    out_specs=_row_spec,
    out_shape=jax.ShapeDtypeStruct((N_PAD, F), jnp.float32),
)

_tc_fin = pl.pallas_call(
    _tc_fin_body,
    grid=(GRID,),
    in_specs=[
        _ag_spec, _row_spec, _degp_spec, _b_spec,
        pl.BlockSpec((BLK, 1), lambda i: (i, 0)),
        pl.BlockSpec((F, 10), lambda i: (0, 0)),
        pl.BlockSpec((1, 10), lambda i: (0, 0)),
    ],
    out_specs=pl.BlockSpec((NG, 10), lambda i: (0, 0)),
    out_shape=jax.ShapeDtypeStruct((NG, 10), jnp.float32),
    scratch_shapes=[
        pltpu.VMEM((NG, F), jnp.float32),
        pltpu.VMEM((NG, F), jnp.float32),
    ],
)


def kernel(x, edge_index, batch, W1, b1, W2, b2, Wlin, blin):
    src = edge_index[0].astype(jnp.int32)
    dst = edge_index[1].astype(jnp.int32)
    pad_e = E_PAD - E_RAW
    srcp = jnp.concatenate([src, jnp.full((pad_e,), N_RAW, jnp.int32)])
    dstp = jnp.concatenate([dst, jnp.full((pad_e,), N_RAW, jnp.int32)])
    dstp2 = dstp.reshape(E_PAD // CH, CH)
    xp = jnp.concatenate(
        [x, jnp.zeros((N_PAD - N_RAW, F), jnp.float32)], axis=0)
    batchp = jnp.concatenate(
        [batch.astype(jnp.int32), jnp.full((N_PAD - N_RAW,), NG, jnp.int32)]
    ).reshape(N_PAD, 1)
    zeros128 = jnp.zeros((N_PAD, F), jnp.float32)
    zeros16 = jnp.zeros((N_PAD, 16), jnp.float32)

    degp = _sc_deg(dstp2, zeros16)
    g1 = _tc_in(xp, degp, W1)
    ag1 = _sc_agg(srcp, dstp, g1, zeros128)
    g2 = _tc_mid(ag1, g1, degp, b1.reshape(1, F), W2)
    ag2 = _sc_agg(srcp, dstp, g2, zeros128)
    logits = _tc_fin(ag2, g2, degp, b2.reshape(1, F), batchp,
                     Wlin, blin.reshape(1, 10))
    return logits


# speedup vs baseline: 1.1321x; 1.1321x over previous
import functools

import jax
import jax.numpy as jnp
from jax import lax
from jax.experimental import pallas as pl
from jax.experimental.pallas import tpu as pltpu
from jax.experimental.pallas import tpu_sc as plsc

N_RAW = 10000
N_PAD = 10240
E_RAW = 320000
CH = 128
N_TILES = 32
BCH = 16
CPT_A = 112
CPT_B = 48
CPT = (CPT_A + CPT_B) // 2
E_PAD = (CPT_A + CPT_B) * 16 * CH
ROWS_PER_TILE = N_PAD // 16
NG = 64
F = 128
BLK = 1024
GRID = N_PAD // BLK

_mesh = plsc.VectorSubcoreMesh(core_axis_name="c", subcore_axis_name="s")


@functools.partial(
    pl.kernel,
    mesh=_mesh,
    out_type=jax.ShapeDtypeStruct((2, N_PAD, 16), jnp.float32),
    scratch_types=[
        pltpu.VMEM((CPT, CH), jnp.int32),
        pltpu.VMEM((CH, 16), jnp.float32),
        pltpu.VMEM_SHARED((N_PAD, 16), jnp.float32),
    ],
)
def _sc_deg(dst_hbm, zeros16_hbm, out_hbm, dbuf, ones_v, acc):
    cid = lax.axis_index("c")
    sid = lax.axis_index("s")
    wid = sid * 2 + cid

    def fill(r, _):
        ones_v[r, :] = jnp.ones((16,), jnp.float32)
        return 0

    lax.fori_loop(0, CH, fill, 0)

    r0 = sid * ROWS_PER_TILE
    pltpu.sync_copy(zeros16_hbm.at[pl.ds(r0, ROWS_PER_TILE)],
                    acc.at[pl.ds(r0, ROWS_PER_TILE)])
    pltpu.sync_copy(dst_hbm.at[pl.ds(wid * CPT, CPT)], dbuf)
    plsc.subcore_barrier()

    def chunk(j, _):
        pltpu.sync_copy(ones_v, acc.at[dbuf.at[j]], add=True)
        return 0

    lax.fori_loop(0, CPT, chunk, 0)
    plsc.subcore_barrier()

    pltpu.sync_copy(acc.at[pl.ds(r0, ROWS_PER_TILE)],
                    out_hbm.at[cid, pl.ds(r0, ROWS_PER_TILE)])


@functools.partial(
    pl.kernel,
    mesh=_mesh,
    out_type=jax.ShapeDtypeStruct((2, N_PAD, F), jnp.float32),
    scratch_types=[
        pltpu.VMEM((BCH, CH), jnp.int32),
        pltpu.VMEM((BCH, CH), jnp.int32),
        pltpu.VMEM((CH, F), jnp.float32),
        pltpu.VMEM((CH, F), jnp.float32),
        pltpu.VMEM_SHARED((N_PAD, F), jnp.float32),
        pltpu.SemaphoreType.DMA,
        pltpu.SemaphoreType.DMA,
    ],
)
def _sc_agg(src_hbm, dst_hbm, g_hbm, zeros_hbm, out_hbm,
            sbuf, dbuf, rows0, rows1, acc, sem0, sem1):
    cid = lax.axis_index("c")
    sid = lax.axis_index("s")

    r0 = sid * ROWS_PER_TILE
    pltpu.sync_copy(zeros_hbm.at[pl.ds(r0, ROWS_PER_TILE)],
                    acc.at[pl.ds(r0, ROWS_PER_TILE)])
    plsc.subcore_barrier()

    nblk = jnp.where(cid == 0, CPT_A // BCH, CPT_B // BCH)
    base_chunk = jnp.where(cid == 0, sid * CPT_A,
                           16 * CPT_A + sid * CPT_B)

    def block(blk, _):
        c0 = base_chunk + blk * BCH
        pltpu.sync_copy(src_hbm.at[pl.ds(c0, BCH)], sbuf)
        pltpu.sync_copy(dst_hbm.at[pl.ds(c0, BCH)], dbuf)

        pltpu.async_copy(g_hbm.at[sbuf.at[0]], rows0, sem0)
        pltpu.async_copy(g_hbm.at[sbuf.at[1]], rows1, sem1)

        def pair(k, _):
            j0 = k * 2
            j1 = j0 + 1
            pltpu.make_async_copy(g_hbm.at[sbuf.at[j0]], rows0, sem0).wait()
            pltpu.sync_copy(rows0, acc.at[dbuf.at[j0]], add=True)

            @pl.when(j0 + 2 < BCH)
            def _():
                pltpu.async_copy(g_hbm.at[sbuf.at[j0 + 2]], rows0, sem0)

            pltpu.make_async_copy(g_hbm.at[sbuf.at[j1]], rows1, sem1).wait()
            pltpu.sync_copy(rows1, acc.at[dbuf.at[j1]], add=True)

            @pl.when(j1 + 2 < BCH)
            def _():
                pltpu.async_copy(g_hbm.at[sbuf.at[j1 + 2]], rows1, sem1)

            return 0

        lax.fori_loop(0, BCH // 2, pair, 0)
        return 0

    lax.fori_loop(0, nblk, block, 0)
    plsc.subcore_barrier()

    pltpu.sync_copy(acc.at[pl.ds(r0, ROWS_PER_TILE)],
                    out_hbm.at[cid, pl.ds(r0, ROWS_PER_TILE)])


def _dinv_of(degp_blk):
    deg = degp_blk[0] + degp_blk[1] + 1.0
    return lax.rsqrt(deg[:, :1])


def _tc_in_body(x_ref, degp_ref, w_ref, g_ref):
    dinv = _dinv_of(degp_ref[...])
    h = jnp.dot(x_ref[...], w_ref[...], preferred_element_type=jnp.float32)
    g_ref[...] = h * dinv


def _tc_mid_body(ag_ref, g_ref, degp_ref, b_ref, w_ref, out_ref):
    dinv = _dinv_of(degp_ref[...])
    ag = ag_ref[...]
    a = (ag[0] + ag[1] + g_ref[...]) * dinv + b_ref[...]
    h = jnp.maximum(a, 0.0)
    out_ref[...] = jnp.dot(h, w_ref[...],
                           preferred_element_type=jnp.float32) * dinv


def _tc_fin_body(ag_ref, g_ref, degp_ref, b_ref, batch_ref, wl_ref, bl_ref,
                 out_ref, acc_s, acc_c):
    i = pl.program_id(0)

    @pl.when(i == 0)
    def _():
        acc_s[...] = jnp.zeros_like(acc_s)
        acc_c[...] = jnp.zeros_like(acc_c)

    dinv = _dinv_of(degp_ref[...])
    ag = ag_ref[...]
    a = (ag[0] + ag[1] + g_ref[...]) * dinv + b_ref[...]
    h = jnp.maximum(a, 0.0)
    gids = lax.broadcasted_iota(jnp.int32, (BLK, NG), 1)
    onehot = (batch_ref[...] == gids).astype(jnp.float32)
    acc_s[...] = acc_s[...] + lax.dot_general(
        onehot, h, (((0,), (0,)), ((), ())),
        preferred_element_type=jnp.float32)
    acc_c[...] = acc_c[...] + jnp.sum(onehot, axis=0)[:, None]

    @pl.when(i == GRID - 1)
    def _():
        pooled = acc_s[...] / jnp.maximum(acc_c[...], 1.0)
        out_ref[...] = jnp.dot(pooled, wl_ref[...],
                               preferred_element_type=jnp.float32) + bl_ref[...]


_degp_spec = pl.BlockSpec((2, BLK, 16), lambda i: (0, i, 0))
_row_spec = pl.BlockSpec((BLK, F), lambda i: (i, 0))
_ag_spec = pl.BlockSpec((2, BLK, F), lambda i: (0, i, 0))
_w_spec = pl.BlockSpec((F, F), lambda i: (0, 0))
_b_spec = pl.BlockSpec((1, F), lambda i: (0, 0))

_tc_in = pl.pallas_call(
    _tc_in_body,
    grid=(GRID,),
    in_specs=[_row_spec, _degp_spec, _w_spec],
    out_specs=_row_spec,
    out_shape=jax.ShapeDtypeStruct((N_PAD, F), jnp.float32),
)

_tc_mid = pl.pallas_call(
    _tc_mid_body,
    grid=(GRID,),
    in_specs=[_ag_spec, _row_spec, _degp_spec, _b_spec, _w_spec],
    out_specs=_row_spec,
    out_shape=jax.ShapeDtypeStruct((N_PAD, F), jnp.float32),
)

_tc_fin = pl.pallas_call(
    _tc_fin_body,
    grid=(GRID,),
    in_specs=[
        _ag_spec, _row_spec, _degp_spec, _b_spec,
        pl.BlockSpec((BLK, 1), lambda i: (i, 0)),
        pl.BlockSpec((F, 10), lambda i: (0, 0)),
        pl.BlockSpec((1, 10), lambda i: (0, 0)),
    ],
    out_specs=pl.BlockSpec((NG, 10), lambda i: (0, 0)),
    out_shape=jax.ShapeDtypeStruct((NG, 10), jnp.float32),
    scratch_shapes=[
        pltpu.VMEM((NG, F), jnp.float32),
        pltpu.VMEM((NG, F), jnp.float32),
    ],
)


def kernel(x, edge_index, batch, W1, b1, W2, b2, Wlin, blin):
    src = edge_index[0].astype(jnp.int32)
    dst = edge_index[1].astype(jnp.int32)
    pad_e = E_PAD - E_RAW
    srcp2 = jnp.concatenate(
        [src, jnp.full((pad_e,), N_RAW, jnp.int32)]).reshape(E_PAD // CH, CH)
    dstp2 = jnp.concatenate(
        [dst, jnp.full((pad_e,), N_RAW, jnp.int32)]).reshape(E_PAD // CH, CH)
    xp = jnp.concatenate(
        [x, jnp.zeros((N_PAD - N_RAW, F), jnp.float32)], axis=0)
    batchp = jnp.concatenate(
        [batch.astype(jnp.int32), jnp.full((N_PAD - N_RAW,), NG, jnp.int32)]
    ).reshape(N_PAD, 1)
    zeros128 = jnp.zeros((N_PAD, F), jnp.float32)
    zeros16 = jnp.zeros((N_PAD, 16), jnp.float32)

    degp = _sc_deg(dstp2, zeros16)
    g1 = _tc_in(xp, degp, W1)
    ag1 = _sc_agg(srcp2, dstp2, g1, zeros128)
    g2 = _tc_mid(ag1, g1, degp, b1.reshape(1, F), W2)
    ag2 = _sc_agg(srcp2, dstp2, g2, zeros128)
    logits = _tc_fin(ag2, g2, degp, b2.reshape(1, F), batchp,
                     Wlin, blin.reshape(1, 10))
    return logits


# speedup vs baseline: 1.1708x; 1.0341x over previous
import functools

import jax
import jax.numpy as jnp
from jax import lax
from jax.experimental import pallas as pl
from jax.experimental.pallas import tpu as pltpu
from jax.experimental.pallas import tpu_sc as plsc

N_RAW = 10000
N_PAD = 10240
E_RAW = 320000
CH = 128
N_TILES = 32
BCH = 32
CPT_A = 128
CPT_B = 32
CPT = (CPT_A + CPT_B) // 2
E_PAD = (CPT_A + CPT_B) * 16 * CH
ROWS_PER_TILE = N_PAD // 16
NG = 64
F = 128
BLK = 1024
GRID = N_PAD // BLK

_mesh = plsc.VectorSubcoreMesh(core_axis_name="c", subcore_axis_name="s")


@functools.partial(
    pl.kernel,
    mesh=_mesh,
    out_type=jax.ShapeDtypeStruct((2, N_PAD, 16), jnp.float32),
    scratch_types=[
        pltpu.VMEM((CPT, CH), jnp.int32),
        pltpu.VMEM((CH, 16), jnp.float32),
        pltpu.VMEM_SHARED((N_PAD, 16), jnp.float32),
    ],
)
def _sc_deg(dst_hbm, zeros16_hbm, out_hbm, dbuf, ones_v, acc):
    cid = lax.axis_index("c")
    sid = lax.axis_index("s")
    wid = sid * 2 + cid

    def fill(r, _):
        ones_v[r, :] = jnp.ones((16,), jnp.float32)
        return 0

    lax.fori_loop(0, CH, fill, 0)

    r0 = sid * ROWS_PER_TILE
    pltpu.sync_copy(zeros16_hbm.at[pl.ds(r0, ROWS_PER_TILE)],
                    acc.at[pl.ds(r0, ROWS_PER_TILE)])
    pltpu.sync_copy(dst_hbm.at[pl.ds(wid * CPT, CPT)], dbuf)
    plsc.subcore_barrier()

    def chunk(j, _):
        pltpu.sync_copy(ones_v, acc.at[dbuf.at[j]], add=True)
        return 0

    lax.fori_loop(0, CPT, chunk, 0)
    plsc.subcore_barrier()

    pltpu.sync_copy(acc.at[pl.ds(r0, ROWS_PER_TILE)],
                    out_hbm.at[cid, pl.ds(r0, ROWS_PER_TILE)])


@functools.partial(
    pl.kernel,
    mesh=_mesh,
    out_type=jax.ShapeDtypeStruct((2, N_PAD, F), jnp.float32),
    scratch_types=[
        pltpu.VMEM((BCH, CH), jnp.int32),
        pltpu.VMEM((BCH, CH), jnp.int32),
        pltpu.VMEM((CH, F), jnp.float32),
        pltpu.VMEM((CH, F), jnp.float32),
        pltpu.VMEM_SHARED((N_PAD, F), jnp.float32),
        pltpu.SemaphoreType.DMA,
        pltpu.SemaphoreType.DMA,
        pltpu.SemaphoreType.DMA,
        pltpu.SemaphoreType.DMA,
    ],
)
def _sc_agg(src_hbm, dst_hbm, g_hbm, zeros_hbm, out_hbm,
            sbuf, dbuf, rows0, rows1, acc, sem0, sem1, sems0, sems1):
    cid = lax.axis_index("c")
    sid = lax.axis_index("s")

    r0 = sid * ROWS_PER_TILE
    pltpu.sync_copy(zeros_hbm.at[pl.ds(r0, ROWS_PER_TILE)],
                    acc.at[pl.ds(r0, ROWS_PER_TILE)])
    plsc.subcore_barrier()

    nblk = jnp.where(cid == 0, CPT_A // BCH, CPT_B // BCH)
    base_chunk = jnp.where(cid == 0, sid * CPT_A,
                           16 * CPT_A + sid * CPT_B)

    def block(blk, _):
        c0 = base_chunk + blk * BCH
        pltpu.sync_copy(src_hbm.at[pl.ds(c0, BCH)], sbuf)
        pltpu.sync_copy(dst_hbm.at[pl.ds(c0, BCH)], dbuf)

        pltpu.async_copy(g_hbm.at[sbuf.at[0]], rows0, sem0)
        pltpu.async_copy(g_hbm.at[sbuf.at[1]], rows1, sem1)

        def pair(k, _):
            j0 = k * 2
            j1 = j0 + 1
            pltpu.make_async_copy(g_hbm.at[sbuf.at[j0]], rows0, sem0).wait()
            pltpu.async_copy(rows0, acc.at[dbuf.at[j0]], sems0, add=True)

            pltpu.make_async_copy(g_hbm.at[sbuf.at[j1]], rows1, sem1).wait()
            pltpu.async_copy(rows1, acc.at[dbuf.at[j1]], sems1, add=True)

            @pl.when(j0 + 2 < BCH)
            def _():
                pltpu.make_async_copy(
                    rows0, acc.at[dbuf.at[j0]], sems0).wait()
                pltpu.async_copy(g_hbm.at[sbuf.at[j0 + 2]], rows0, sem0)

            @pl.when(j1 + 2 < BCH)
            def _():
                pltpu.make_async_copy(
                    rows1, acc.at[dbuf.at[j1]], sems1).wait()
                pltpu.async_copy(g_hbm.at[sbuf.at[j1 + 2]], rows1, sem1)

            return 0

        lax.fori_loop(0, BCH // 2, pair, 0)

        pltpu.make_async_copy(rows0, acc.at[dbuf.at[BCH - 2]], sems0).wait()
        pltpu.make_async_copy(rows1, acc.at[dbuf.at[BCH - 1]], sems1).wait()
        return 0

    lax.fori_loop(0, nblk, block, 0)
    plsc.subcore_barrier()

    pltpu.sync_copy(acc.at[pl.ds(r0, ROWS_PER_TILE)],
                    out_hbm.at[cid, pl.ds(r0, ROWS_PER_TILE)])


def _dinv_of(degp_blk):
    deg = degp_blk[0] + degp_blk[1] + 1.0
    return lax.rsqrt(deg[:, :1])


def _tc_in_body(x_ref, degp_ref, w_ref, g_ref):
    dinv = _dinv_of(degp_ref[...])
    h = jnp.dot(x_ref[...], w_ref[...], preferred_element_type=jnp.float32)
    g_ref[...] = h * dinv


def _tc_mid_body(ag_ref, g_ref, degp_ref, b_ref, w_ref, out_ref):
    dinv = _dinv_of(degp_ref[...])
    ag = ag_ref[...]
    a = (ag[0] + ag[1] + g_ref[...]) * dinv + b_ref[...]
    h = jnp.maximum(a, 0.0)
    out_ref[...] = jnp.dot(h, w_ref[...],
                           preferred_element_type=jnp.float32) * dinv


def _tc_fin_body(ag_ref, g_ref, degp_ref, b_ref, batch_ref, wl_ref, bl_ref,
                 out_ref, acc_s, acc_c):
    i = pl.program_id(0)

    @pl.when(i == 0)
    def _():
        acc_s[...] = jnp.zeros_like(acc_s)
        acc_c[...] = jnp.zeros_like(acc_c)

    dinv = _dinv_of(degp_ref[...])
    ag = ag_ref[...]
    a = (ag[0] + ag[1] + g_ref[...]) * dinv + b_ref[...]
    h = jnp.maximum(a, 0.0)
    gids = lax.broadcasted_iota(jnp.int32, (BLK, NG), 1)
    onehot = (batch_ref[...] == gids).astype(jnp.float32)
    acc_s[...] = acc_s[...] + lax.dot_general(
        onehot, h, (((0,), (0,)), ((), ())),
        preferred_element_type=jnp.float32)
    acc_c[...] = acc_c[...] + jnp.sum(onehot, axis=0)[:, None]

    @pl.when(i == GRID - 1)
    def _():
        pooled = acc_s[...] / jnp.maximum(acc_c[...], 1.0)
        out_ref[...] = jnp.dot(pooled, wl_ref[...],
                               preferred_element_type=jnp.float32) + bl_ref[...]


_degp_spec = pl.BlockSpec((2, BLK, 16), lambda i: (0, i, 0))
_row_spec = pl.BlockSpec((BLK, F), lambda i: (i, 0))
_ag_spec = pl.BlockSpec((2, BLK, F), lambda i: (0, i, 0))
_w_spec = pl.BlockSpec((F, F), lambda i: (0, 0))
_b_spec = pl.BlockSpec((1, F), lambda i: (0, 0))

_tc_in = pl.pallas_call(
    _tc_in_body,
    grid=(GRID,),
    in_specs=[_row_spec, _degp_spec, _w_spec],
    out_specs=_row_spec,
    out_shape=jax.ShapeDtypeStruct((N_PAD, F), jnp.float32),
)

_tc_mid = pl.pallas_call(
    _tc_mid_body,
    grid=(GRID,),
    in_specs=[_ag_spec, _row_spec, _degp_spec, _b_spec, _w_spec],
    out_specs=_row_spec,
    out_shape=jax.ShapeDtypeStruct((N_PAD, F), jnp.float32),
)

_tc_fin = pl.pallas_call(
    _tc_fin_body,
    grid=(GRID,),
    in_specs=[
        _ag_spec, _row_spec, _degp_spec, _b_spec,
        pl.BlockSpec((BLK, 1), lambda i: (i, 0)),
        pl.BlockSpec((F, 10), lambda i: (0, 0)),
        pl.BlockSpec((1, 10), lambda i: (0, 0)),
    ],
    out_specs=pl.BlockSpec((NG, 10), lambda i: (0, 0)),
    out_shape=jax.ShapeDtypeStruct((NG, 10), jnp.float32),
    scratch_shapes=[
        pltpu.VMEM((NG, F), jnp.float32),
        pltpu.VMEM((NG, F), jnp.float32),
    ],
)


def kernel(x, edge_index, batch, W1, b1, W2, b2, Wlin, blin):
    src = edge_index[0].astype(jnp.int32)
    dst = edge_index[1].astype(jnp.int32)
    pad_e = E_PAD - E_RAW
    srcp2 = jnp.concatenate(
        [src, jnp.full((pad_e,), N_RAW, jnp.int32)]).reshape(E_PAD // CH, CH)
    dstp2 = jnp.concatenate(
        [dst, jnp.full((pad_e,), N_RAW, jnp.int32)]).reshape(E_PAD // CH, CH)
    xp = jnp.concatenate(
        [x, jnp.zeros((N_PAD - N_RAW, F), jnp.float32)], axis=0)
    batchp = jnp.concatenate(
        [batch.astype(jnp.int32), jnp.full((N_PAD - N_RAW,), NG, jnp.int32)]
    ).reshape(N_PAD, 1)
    zeros128 = jnp.zeros((N_PAD, F), jnp.float32)
    zeros16 = jnp.zeros((N_PAD, 16), jnp.float32)

    degp = _sc_deg(dstp2, zeros16)
    g1 = _tc_in(xp, degp, W1)
    ag1 = _sc_agg(srcp2, dstp2, g1, zeros128)
    g2 = _tc_mid(ag1, g1, degp, b1.reshape(1, F), W2)
    ag2 = _sc_agg(srcp2, dstp2, g2, zeros128)
    logits = _tc_fin(ag2, g2, degp, b2.reshape(1, F), batchp,
                     Wlin, blin.reshape(1, 10))
    return logits


# speedup vs baseline: 1.3460x; 1.1497x over previous
import functools

import jax
import jax.numpy as jnp
from jax import lax
from jax.experimental import pallas as pl
from jax.experimental.pallas import tpu as pltpu
from jax.experimental.pallas import tpu_sc as plsc

N_RAW = 10000
N_PAD = 10240
E_RAW = 320000
CH = 128
N_TILES = 32
BCH = 16
CPT_A = 144
CPT_B = 16
CPT = (CPT_A + CPT_B) // 2
E_PAD = (CPT_A + CPT_B) * 16 * CH
ROWS_PER_TILE = N_PAD // 16
NG = 64
F = 128
BLK = 1024
GRID = N_PAD // BLK

_mesh = plsc.VectorSubcoreMesh(core_axis_name="c", subcore_axis_name="s")


@functools.partial(
    pl.kernel,
    mesh=_mesh,
    out_type=jax.ShapeDtypeStruct((2, N_PAD, 16), jnp.float32),
    scratch_types=[
        pltpu.VMEM((CPT, CH), jnp.int32),
        pltpu.VMEM((CH, 16), jnp.float32),
        pltpu.VMEM_SHARED((N_PAD, 16), jnp.float32),
    ],
)
def _sc_deg(dst_hbm, zeros16_hbm, out_hbm, dbuf, ones_v, acc):
    cid = lax.axis_index("c")
    sid = lax.axis_index("s")
    wid = sid * 2 + cid

    def fill(r, _):
        ones_v[r, :] = jnp.ones((16,), jnp.float32)
        return 0

    lax.fori_loop(0, CH, fill, 0)

    r0 = sid * ROWS_PER_TILE
    pltpu.sync_copy(zeros16_hbm.at[pl.ds(r0, ROWS_PER_TILE)],
                    acc.at[pl.ds(r0, ROWS_PER_TILE)])
    pltpu.sync_copy(dst_hbm.at[pl.ds(wid * CPT, CPT)], dbuf)
    plsc.subcore_barrier()

    def chunk(j, _):
        pltpu.sync_copy(ones_v, acc.at[dbuf.at[j]], add=True)
        return 0

    lax.fori_loop(0, CPT, chunk, 0)
    plsc.subcore_barrier()

    pltpu.sync_copy(acc.at[pl.ds(r0, ROWS_PER_TILE)],
                    out_hbm.at[cid, pl.ds(r0, ROWS_PER_TILE)])


@functools.partial(
    pl.kernel,
    mesh=_mesh,
    out_type=jax.ShapeDtypeStruct((2, N_PAD, F), jnp.float32),
    scratch_types=[
        pltpu.VMEM((BCH, CH), jnp.int32),
        pltpu.VMEM((BCH, CH), jnp.int32),
        pltpu.VMEM((CH, F), jnp.float32),
        pltpu.VMEM((CH, F), jnp.float32),
        pltpu.VMEM_SHARED((N_PAD, F), jnp.float32),
        pltpu.SemaphoreType.DMA,
        pltpu.SemaphoreType.DMA,
        pltpu.SemaphoreType.DMA,
        pltpu.SemaphoreType.DMA,
    ],
)
def _sc_agg(src_hbm, dst_hbm, g_hbm, zeros_hbm, out_hbm,
            sbuf, dbuf, rows0, rows1, acc, sem0, sem1, sems0, sems1):
    cid = lax.axis_index("c")
    sid = lax.axis_index("s")

    r0 = sid * ROWS_PER_TILE
    pltpu.sync_copy(zeros_hbm.at[pl.ds(r0, ROWS_PER_TILE)],
                    acc.at[pl.ds(r0, ROWS_PER_TILE)])
    plsc.subcore_barrier()

    nblk = jnp.where(cid == 0, CPT_A // BCH, CPT_B // BCH)
    base_chunk = jnp.where(cid == 0, sid * CPT_A,
                           16 * CPT_A + sid * CPT_B)

    def block(blk, _):
        c0 = base_chunk + blk * BCH
        pltpu.sync_copy(src_hbm.at[pl.ds(c0, BCH)], sbuf)
        pltpu.sync_copy(dst_hbm.at[pl.ds(c0, BCH)], dbuf)

        pltpu.async_copy(g_hbm.at[sbuf.at[0]], rows0, sem0)
        pltpu.async_copy(g_hbm.at[sbuf.at[1]], rows1, sem1)

        def pair(k, _):
            j0 = k * 2
            j1 = j0 + 1
            pltpu.make_async_copy(g_hbm.at[sbuf.at[j0]], rows0, sem0).wait()
            pltpu.async_copy(rows0, acc.at[dbuf.at[j0]], sems0, add=True)

            pltpu.make_async_copy(g_hbm.at[sbuf.at[j1]], rows1, sem1).wait()
            pltpu.async_copy(rows1, acc.at[dbuf.at[j1]], sems1, add=True)

            @pl.when(j0 + 2 < BCH)
            def _():
                pltpu.make_async_copy(
                    rows0, acc.at[dbuf.at[j0]], sems0).wait()
                pltpu.async_copy(g_hbm.at[sbuf.at[j0 + 2]], rows0, sem0)

            @pl.when(j1 + 2 < BCH)
            def _():
                pltpu.make_async_copy(
                    rows1, acc.at[dbuf.at[j1]], sems1).wait()
                pltpu.async_copy(g_hbm.at[sbuf.at[j1 + 2]], rows1, sem1)

            return 0

        lax.fori_loop(0, BCH // 2, pair, 0)

        pltpu.make_async_copy(rows0, acc.at[dbuf.at[BCH - 2]], sems0).wait()
        pltpu.make_async_copy(rows1, acc.at[dbuf.at[BCH - 1]], sems1).wait()
        return 0

    lax.fori_loop(0, nblk, block, 0)
    plsc.subcore_barrier()

    pltpu.sync_copy(acc.at[pl.ds(r0, ROWS_PER_TILE)],
                    out_hbm.at[cid, pl.ds(r0, ROWS_PER_TILE)])


def _dinv_of(degp_blk):
    deg = degp_blk[0] + degp_blk[1] + 1.0
    return lax.rsqrt(deg[:, :1])


def _tc_in_body(x_ref, degp_ref, w_ref, g_ref):
    dinv = _dinv_of(degp_ref[...])
    h = jnp.dot(x_ref[...], w_ref[...], preferred_element_type=jnp.float32)
    g_ref[...] = h * dinv


def _tc_mid_body(ag_ref, g_ref, degp_ref, b_ref, w_ref, out_ref):
    dinv = _dinv_of(degp_ref[...])
    ag = ag_ref[...]
    a = (ag[0] + ag[1] + g_ref[...]) * dinv + b_ref[...]
    h = jnp.maximum(a, 0.0)
    out_ref[...] = jnp.dot(h, w_ref[...],
                           preferred_element_type=jnp.float32) * dinv


def _tc_fin_body(ag_ref, g_ref, degp_ref, b_ref, batch_ref, wl_ref, bl_ref,
                 out_ref, acc_s, acc_c):
    i = pl.program_id(0)

    @pl.when(i == 0)
    def _():
        acc_s[...] = jnp.zeros_like(acc_s)
        acc_c[...] = jnp.zeros_like(acc_c)

    dinv = _dinv_of(degp_ref[...])
    ag = ag_ref[...]
    a = (ag[0] + ag[1] + g_ref[...]) * dinv + b_ref[...]
    h = jnp.maximum(a, 0.0)
    gids = lax.broadcasted_iota(jnp.int32, (BLK, NG), 1)
    onehot = (batch_ref[...] == gids).astype(jnp.float32)
    acc_s[...] = acc_s[...] + lax.dot_general(
        onehot, h, (((0,), (0,)), ((), ())),
        preferred_element_type=jnp.float32)
    acc_c[...] = acc_c[...] + jnp.sum(onehot, axis=0)[:, None]

    @pl.when(i == GRID - 1)
    def _():
        pooled = acc_s[...] / jnp.maximum(acc_c[...], 1.0)
        out_ref[...] = jnp.dot(pooled, wl_ref[...],
                               preferred_element_type=jnp.float32) + bl_ref[...]


_degp_spec = pl.BlockSpec((2, BLK, 16), lambda i: (0, i, 0))
_row_spec = pl.BlockSpec((BLK, F), lambda i: (i, 0))
_ag_spec = pl.BlockSpec((2, BLK, F), lambda i: (0, i, 0))
_w_spec = pl.BlockSpec((F, F), lambda i: (0, 0))
_b_spec = pl.BlockSpec((1, F), lambda i: (0, 0))

_tc_in = pl.pallas_call(
    _tc_in_body,
    grid=(GRID,),
    in_specs=[_row_spec, _degp_spec, _w_spec],
    out_specs=_row_spec,
    out_shape=jax.ShapeDtypeStruct((N_PAD, F), jnp.float32),
)

_tc_mid = pl.pallas_call(
    _tc_mid_body,
    grid=(GRID,),
    in_specs=[_ag_spec, _row_spec, _degp_spec, _b_spec, _w_spec],
    out_specs=_row_spec,
    out_shape=jax.ShapeDtypeStruct((N_PAD, F), jnp.float32),
)

_tc_fin = pl.pallas_call(
    _tc_fin_body,
    grid=(GRID,),
    in_specs=[
        _ag_spec, _row_spec, _degp_spec, _b_spec,
        pl.BlockSpec((BLK, 1), lambda i: (i, 0)),
        pl.BlockSpec((F, 10), lambda i: (0, 0)),
        pl.BlockSpec((1, 10), lambda i: (0, 0)),
    ],
    out_specs=pl.BlockSpec((NG, 10), lambda i: (0, 0)),
    out_shape=jax.ShapeDtypeStruct((NG, 10), jnp.float32),
    scratch_shapes=[
        pltpu.VMEM((NG, F), jnp.float32),
        pltpu.VMEM((NG, F), jnp.float32),
    ],
)


def kernel(x, edge_index, batch, W1, b1, W2, b2, Wlin, blin):
    src = edge_index[0].astype(jnp.int32)
    dst = edge_index[1].astype(jnp.int32)
    pad_e = E_PAD - E_RAW
    srcp2 = jnp.concatenate(
        [src, jnp.full((pad_e,), N_RAW, jnp.int32)]).reshape(E_PAD // CH, CH)
    dstp2 = jnp.concatenate(
        [dst, jnp.full((pad_e,), N_RAW, jnp.int32)]).reshape(E_PAD // CH, CH)
    xp = jnp.concatenate(
        [x, jnp.zeros((N_PAD - N_RAW, F), jnp.float32)], axis=0)
    batchp = jnp.concatenate(
        [batch.astype(jnp.int32), jnp.full((N_PAD - N_RAW,), NG, jnp.int32)]
    ).reshape(N_PAD, 1)
    zeros128 = jnp.zeros((N_PAD, F), jnp.float32)
    zeros16 = jnp.zeros((N_PAD, 16), jnp.float32)

    degp = _sc_deg(dstp2, zeros16)
    g1 = _tc_in(xp, degp, W1)
    ag1 = _sc_agg(srcp2, dstp2, g1, zeros128)
    g2 = _tc_mid(ag1, g1, degp, b1.reshape(1, F), W2)
    ag2 = _sc_agg(srcp2, dstp2, g2, zeros128)
    logits = _tc_fin(ag2, g2, degp, b2.reshape(1, F), batchp,
                     Wlin, blin.reshape(1, 10))
    return logits
